# baseline (speedup 1.0000x reference)
import functools

import jax
import jax.numpy as jnp
from jax import lax
from jax.experimental import pallas as pl
from jax.experimental.pallas import tpu as pltpu
from jax.experimental.pallas import tpu_sc as plsc

_N = 16384
_D = 128
_C = 100
_CP = 128
_B = 2048
_K = _N // _B
_NW = 32
_RW = _N // _NW
_CW = 144
_ACC = _C * _CW


def _sc_body(feat_hbm, lab_hbm, out_hbm, f_v, lab_v, acc_v):
    wid = lax.axis_index("s") * 2 + lax.axis_index("c")
    base = wid * _RW
    pltpu.sync_copy(feat_hbm.at[pl.ds(base, _RW), :], f_v)
    pltpu.sync_copy(lab_hbm.at[pl.ds(base, _RW)], lab_v)

    zeros16 = jnp.zeros((16,), jnp.float32)

    def _zero(t, c):
        for u in range(10):
            acc_v[pl.ds(t * 160 + u * 16, 16)] = zeros16
        return c
    lax.fori_loop(0, _ACC // 160, _zero, 0)

    lane = lax.broadcasted_iota(jnp.int32, (16,), 0)
    ones16 = jnp.ones((16,), jnp.float32)
    mask0 = lane == 0

    def _group(g, c):
        lvec = lab_v[pl.ds(g * 16, 16)]
        for j in range(16):
            lj = lax.gather(
                lvec, jnp.full((16, 1), j, jnp.int32),
                lax.GatherDimensionNumbers(offset_dims=(),
                                           collapsed_slice_dims=(0,),
                                           start_index_map=(0,)),
                (1,), mode=lax.GatherScatterMode.PROMISE_IN_BOUNDS)
            bvec = lj * _CW + lane
            row = g * 16 + j
            for k in range(8):
                plsc.addupdate_scatter(
                    acc_v, [bvec + k * 16], f_v[row, pl.ds(k * 16, 16)])
            plsc.addupdate_scatter(acc_v, [bvec + _D], ones16, mask=mask0)
        return c
    lax.fori_loop(0, _RW // 16, _group, 0)

    pltpu.sync_copy(acc_v, out_hbm.at[wid])


_sc_segsum = functools.partial(
    pl.kernel,
    out_type=jax.ShapeDtypeStruct((_NW, _ACC), jnp.float32),
    mesh=plsc.VectorSubcoreMesh(core_axis_name="c", subcore_axis_name="s"),
    compiler_params=pltpu.CompilerParams(needs_layout_passes=False),
    scratch_types=[
        pltpu.VMEM((_RW, _D), jnp.float32),
        pltpu.VMEM((_RW,), jnp.int32),
        pltpu.VMEM((_ACC,), jnp.float32),
    ],
)(_sc_body)


def _tc_body(part_ref, lab_ref, feat_ref, cent_ref, out_ref,
             m_ref, cb_ref, acc_ref, sc_ref):
    i = pl.program_id(0)

    @pl.when(i == 0)
    def _boundary():
        t = jnp.sum(part_ref[...], axis=0)
        cs = t[:, :_D]
        ni = t[:, _D]
        om = jnp.sum(cs, axis=0) / _N
        ni_safe = jnp.where(ni > 0, ni, 1.0)
        mc = cs / ni_safe[:, None]
        om_norm = jnp.sqrt(jnp.sum(om * om))
        mcn = jnp.sqrt(jnp.sum(mc * mc, axis=1))
        mcn_safe = jnp.where(mcn > 0, mcn, 1.0)
        d_cls = 1.0 - jnp.sum(mc * om[None, :], axis=1) / mcn_safe * om_norm
        sc_ref[0] = jnp.sum((ni / _N) * d_cls * d_cls)
        m = mc * mcn[:, None]
        m_ref[...] = jnp.concatenate(
            [m, jnp.zeros((_CP - _C, _D), jnp.float32)], axis=0)
        cent = cent_ref[...]
        cbn = jnp.sqrt(jnp.sum(cent * cent, axis=1))
        cb_ref[...] = cent * cbn[:, None]
        acc_ref[...] = jnp.zeros((4, _CP), jnp.float32)

    @pl.when(i > 0)
    def _phase_b():
        f = feat_ref[...]
        lab = lab_ref[0, 0, :]
        oh = (lab[:, None] == jax.lax.broadcasted_iota(jnp.int32, (_B, _CP), 1)
              ).astype(jnp.float32)
        inv_rn = jax.lax.rsqrt(jnp.sum(f * f, axis=1))
        fn = f * inv_rn[:, None]
        p1 = jax.lax.dot_general(
            fn, m_ref[...], (((1,), (1,)), ((), ())),
            precision=jax.lax.Precision.HIGHEST,
            preferred_element_type=jnp.float32)
        p2 = jax.lax.dot_general(
            fn, cb_ref[...], (((1,), (1,)), ((), ())),
            precision=jax.lax.Precision.HIGHEST,
            preferred_element_type=jnp.float32)
        a1 = oh * p1
        a3 = oh * p2
        part = jnp.concatenate([
            jnp.sum(a1, axis=0, keepdims=True),
            jnp.sum(a1 * p1, axis=0, keepdims=True),
            jnp.sum(a3, axis=0, keepdims=True),
            jnp.sum(a3 * p2, axis=0, keepdims=True),
        ], axis=0)
        acc_ref[...] += part

    @pl.when(i == _K)
    def _final():
        accs = acc_ref[...]
        sw = _N - 2.0 * jnp.sum(accs[0, :]) + jnp.sum(accs[1, :])
        ct = _N - 2.0 * jnp.sum(accs[2, :]) + jnp.sum(accs[3, :])
        loss = (sw / _N) / sc_ref[0] + ct / _N
        out_ref[...] = jnp.full((1, 1), loss, jnp.float32)


def kernel(features, labels, centers):
    part = _sc_segsum(features, labels)
    part3 = part.reshape(_NW, _C, _CW)
    lab3 = labels.reshape(_K, 1, _B)
    cent_p = jnp.pad(centers, ((0, _CP - _C), (0, 0)))
    out = pl.pallas_call(
        _tc_body,
        grid=(_K + 1,),
        in_specs=[
            pl.BlockSpec((_NW, _C, _CW), lambda i: (0, 0, 0)),
            pl.BlockSpec((1, 1, _B), lambda i: (jnp.maximum(i - 1, 0), 0, 0)),
            pl.BlockSpec((_B, _D), lambda i: (jnp.maximum(i - 1, 0), 0)),
            pl.BlockSpec((_CP, _D), lambda i: (0, 0)),
        ],
        out_specs=pl.BlockSpec((1, 1), lambda i: (0, 0)),
        out_shape=jax.ShapeDtypeStruct((1, 1), jnp.float32),
        scratch_shapes=[
            pltpu.VMEM((_CP, _D), jnp.float32),
            pltpu.VMEM((_CP, _D), jnp.float32),
            pltpu.VMEM((4, _CP), jnp.float32),
            pltpu.SMEM((4,), jnp.float32),
        ],
    )(part3, lab3, features, cent_p)
    return out.reshape(())


# speedup vs baseline: 1.2740x; 1.2740x over previous
import functools

import jax
import jax.numpy as jnp
from jax import lax
from jax.experimental import pallas as pl
from jax.experimental.pallas import tpu as pltpu
from jax.experimental.pallas import tpu_sc as plsc

_N = 16384
_D = 128
_C = 100
_CP = 128
_B = 2048
_K = _N // _B
_NW = 32
_RW = _N // _NW
_CW = 144
_ACC = _C * _CW


def _sc_body(feat_hbm, lab_hbm, out_hbm, f_v, lab_v, acc_v):
    wid = lax.axis_index("s") * 2 + lax.axis_index("c")
    base = wid * _RW
    pltpu.sync_copy(feat_hbm.at[pl.ds(base, _RW), :], f_v)
    pltpu.sync_copy(lab_hbm.at[pl.ds(base, _RW)], lab_v)

    zeros16 = jnp.zeros((16,), jnp.float32)

    @plsc.parallel_loop(0, _ACC // 160)
    def _zero(t):
        for u in range(10):
            acc_v[pl.ds(t * 160 + u * 16, 16)] = zeros16

    lane = lax.broadcasted_iota(jnp.int32, (16,), 0)
    ones16 = jnp.ones((16,), jnp.float32)
    mask0 = lane == 0

    @plsc.parallel_loop(0, _RW // 16)
    def _group(g):
        lvec = lab_v[pl.ds(g * 16, 16)]
        for j in range(16):
            lj = lax.gather(
                lvec, jnp.full((16, 1), j, jnp.int32),
                lax.GatherDimensionNumbers(offset_dims=(),
                                           collapsed_slice_dims=(0,),
                                           start_index_map=(0,)),
                (1,), mode=lax.GatherScatterMode.PROMISE_IN_BOUNDS)
            bvec = lj * _CW + lane
            row = g * 16 + j
            for k in range(8):
                plsc.addupdate_scatter(
                    acc_v, [bvec + k * 16], f_v[row, pl.ds(k * 16, 16)])
            plsc.addupdate_scatter(acc_v, [bvec + _D], ones16, mask=mask0)

    pltpu.sync_copy(acc_v, out_hbm.at[wid])


_sc_segsum = functools.partial(
    pl.kernel,
    out_type=jax.ShapeDtypeStruct((_NW, _ACC), jnp.float32),
    mesh=plsc.VectorSubcoreMesh(core_axis_name="c", subcore_axis_name="s"),
    compiler_params=pltpu.CompilerParams(needs_layout_passes=False),
    scratch_types=[
        pltpu.VMEM((_RW, _D), jnp.float32),
        pltpu.VMEM((_RW,), jnp.int32),
        pltpu.VMEM((_ACC,), jnp.float32),
    ],
)(_sc_body)


def _tc_body(part_ref, lab_ref, feat_ref, cent_ref, out_ref,
             m_ref, cb_ref, acc_ref, sc_ref):
    i = pl.program_id(0)

    @pl.when(i == 0)
    def _boundary():
        t = jnp.sum(part_ref[...], axis=0)
        cs = t[:, :_D]
        ni = t[:, _D]
        om = jnp.sum(cs, axis=0) / _N
        ni_safe = jnp.where(ni > 0, ni, 1.0)
        mc = cs / ni_safe[:, None]
        om_norm = jnp.sqrt(jnp.sum(om * om))
        mcn = jnp.sqrt(jnp.sum(mc * mc, axis=1))
        mcn_safe = jnp.where(mcn > 0, mcn, 1.0)
        d_cls = 1.0 - jnp.sum(mc * om[None, :], axis=1) / mcn_safe * om_norm
        sc_ref[0] = jnp.sum((ni / _N) * d_cls * d_cls)
        m = mc * mcn[:, None]
        m_ref[...] = jnp.concatenate(
            [m, jnp.zeros((_CP - _C, _D), jnp.float32)], axis=0)
        cent = cent_ref[...]
        cbn = jnp.sqrt(jnp.sum(cent * cent, axis=1))
        cb_ref[...] = jnp.concatenate(
            [cent * cbn[:, None], jnp.zeros((_CP - _C, _D), jnp.float32)],
            axis=0)
        acc_ref[...] = jnp.zeros((4, _CP), jnp.float32)

    @pl.when(i > 0)
    def _phase_b():
        f = feat_ref[...]
        lab = lab_ref[0, 0, :]
        oh = (lab[:, None] == jax.lax.broadcasted_iota(jnp.int32, (_B, _CP), 1)
              ).astype(jnp.float32)
        inv_rn = jax.lax.rsqrt(jnp.sum(f * f, axis=1))
        fn = f * inv_rn[:, None]
        p1 = jax.lax.dot_general(
            fn, m_ref[...], (((1,), (1,)), ((), ())),
            preferred_element_type=jnp.float32)
        p2 = jax.lax.dot_general(
            fn, cb_ref[...], (((1,), (1,)), ((), ())),
            preferred_element_type=jnp.float32)
        a1 = oh * p1
        a3 = oh * p2
        part = jnp.concatenate([
            jnp.sum(a1, axis=0, keepdims=True),
            jnp.sum(a1 * p1, axis=0, keepdims=True),
            jnp.sum(a3, axis=0, keepdims=True),
            jnp.sum(a3 * p2, axis=0, keepdims=True),
        ], axis=0)
        acc_ref[...] += part

    @pl.when(i == _K)
    def _final():
        accs = acc_ref[...]
        sw = _N - 2.0 * jnp.sum(accs[0, :]) + jnp.sum(accs[1, :])
        ct = _N - 2.0 * jnp.sum(accs[2, :]) + jnp.sum(accs[3, :])
        loss = (sw / _N) / sc_ref[0] + ct / _N
        out_ref[...] = jnp.full((1, 1), loss, jnp.float32)


def kernel(features, labels, centers):
    part = _sc_segsum(features, labels)
    part3 = part.reshape(_NW, _C, _CW)
    lab3 = labels.reshape(_K, 1, _B)
    out = pl.pallas_call(
        _tc_body,
        grid=(_K + 1,),
        in_specs=[
            pl.BlockSpec((_NW, _C, _CW), lambda i: (0, 0, 0)),
            pl.BlockSpec((1, 1, _B), lambda i: (jnp.maximum(i - 1, 0), 0, 0)),
            pl.BlockSpec((_B, _D), lambda i: (jnp.maximum(i - 1, 0), 0)),
            pl.BlockSpec((_C, _D), lambda i: (0, 0)),
        ],
        out_specs=pl.BlockSpec((1, 1), lambda i: (0, 0)),
        out_shape=jax.ShapeDtypeStruct((1, 1), jnp.float32),
        scratch_shapes=[
            pltpu.VMEM((_CP, _D), jnp.float32),
            pltpu.VMEM((_CP, _D), jnp.float32),
            pltpu.VMEM((4, _CP), jnp.float32),
            pltpu.SMEM((4,), jnp.float32),
        ],
    )(part3, lab3, features, centers)
    return out.reshape(())


# speedup vs baseline: 1.3127x; 1.0304x over previous
import functools

import jax
import jax.numpy as jnp
from jax import lax
from jax.experimental import pallas as pl
from jax.experimental.pallas import tpu as pltpu
from jax.experimental.pallas import tpu_sc as plsc

_N = 16384
_D = 128
_C = 100
_CP = 128
_B = 2048
_K = _N // _B
_NW = 32
_RW = _N // _NW
_CW = 144
_ACC = _C * _CW


def _sc_body(feat_hbm, lab_hbm, out_hbm, f_v, lab_v, acc_v):
    wid = lax.axis_index("s") * 2 + lax.axis_index("c")
    base = wid * _RW
    pltpu.sync_copy(feat_hbm.at[pl.ds(base, _RW), :], f_v)
    pltpu.sync_copy(lab_hbm.at[pl.ds(base, _RW)], lab_v)

    zeros16 = jnp.zeros((16,), jnp.float32)

    @plsc.parallel_loop(0, _ACC // 160)
    def _zero(t):
        for u in range(10):
            acc_v[pl.ds(t * 160 + u * 16, 16)] = zeros16

    lane = lax.broadcasted_iota(jnp.int32, (16,), 0)
    ones16 = jnp.ones((16,), jnp.float32)
    mask0 = lane == 0

    @plsc.parallel_loop(0, _RW // 16)
    def _group(g):
        lvec = lab_v[pl.ds(g * 16, 16)]
        for j in range(16):
            lj = lax.gather(
                lvec, jnp.full((16, 1), j, jnp.int32),
                lax.GatherDimensionNumbers(offset_dims=(),
                                           collapsed_slice_dims=(0,),
                                           start_index_map=(0,)),
                (1,), mode=lax.GatherScatterMode.PROMISE_IN_BOUNDS)
            bvec = lj * _CW + lane
            row = g * 16 + j
            for k in range(8):
                plsc.addupdate_scatter(
                    acc_v, [bvec + k * 16], f_v[row, pl.ds(k * 16, 16)])
            plsc.addupdate_scatter(acc_v, [bvec + _D], ones16, mask=mask0)

    pltpu.sync_copy(acc_v, out_hbm.at[wid])


_sc_segsum = functools.partial(
    pl.kernel,
    out_type=jax.ShapeDtypeStruct((_NW, _ACC), jnp.float32),
    mesh=plsc.VectorSubcoreMesh(core_axis_name="c", subcore_axis_name="s"),
    compiler_params=pltpu.CompilerParams(needs_layout_passes=False),
    scratch_types=[
        pltpu.VMEM((_RW, _D), jnp.float32),
        pltpu.VMEM((_RW,), jnp.int32),
        pltpu.VMEM((_ACC,), jnp.float32),
    ],
)(_sc_body)


def _onehot_moments(lab, p):
    oh = (lab[:, None] == jax.lax.broadcasted_iota(jnp.int32, (_B, _CP), 1)
          ).astype(jnp.float32)
    a = oh * p
    return jnp.concatenate([
        jnp.sum(a, axis=0, keepdims=True),
        jnp.sum(a * p, axis=0, keepdims=True),
    ], axis=0)


def _ct_body(lab_ref, feat_ref, cent_ref, out_ref, cb_ref, acc_ref):
    i = pl.program_id(0)

    @pl.when(i == 0)
    def _prep():
        cent = cent_ref[...]
        cbn = jnp.sqrt(jnp.sum(cent * cent, axis=1))
        cb_ref[...] = jnp.concatenate(
            [cent * cbn[:, None], jnp.zeros((_CP - _C, _D), jnp.float32)],
            axis=0)
        acc_ref[...] = jnp.zeros((2, _CP), jnp.float32)

    f = feat_ref[...]
    inv_rn = jax.lax.rsqrt(jnp.sum(f * f, axis=1))
    fn = f * inv_rn[:, None]
    p2 = jax.lax.dot_general(
        fn, cb_ref[...], (((1,), (1,)), ((), ())),
        preferred_element_type=jnp.float32)
    acc_ref[...] += _onehot_moments(lab_ref[0, 0, :], p2)

    @pl.when(i == _K - 1)
    def _emit():
        out_ref[...] = acc_ref[...]


def _fin_body(part_ref, lab_ref, feat_ref, ct_ref, out_ref,
              m_ref, acc_ref, sc_ref):
    i = pl.program_id(0)

    @pl.when(i == 0)
    def _boundary():
        t = jnp.sum(part_ref[...], axis=0)
        cs = t[:, :_D]
        ni = t[:, _D]
        om = jnp.sum(cs, axis=0) / _N
        ni_safe = jnp.where(ni > 0, ni, 1.0)
        mc = cs / ni_safe[:, None]
        om_norm = jnp.sqrt(jnp.sum(om * om))
        mcn = jnp.sqrt(jnp.sum(mc * mc, axis=1))
        mcn_safe = jnp.where(mcn > 0, mcn, 1.0)
        d_cls = 1.0 - jnp.sum(mc * om[None, :], axis=1) / mcn_safe * om_norm
        sc_ref[0] = jnp.sum((ni / _N) * d_cls * d_cls)
        m = mc * mcn[:, None]
        m_ref[...] = jnp.concatenate(
            [m, jnp.zeros((_CP - _C, _D), jnp.float32)], axis=0)
        acc_ref[...] = jnp.zeros((2, _CP), jnp.float32)

    @pl.when(i > 0)
    def _phase_b():
        f = feat_ref[...]
        inv_rn = jax.lax.rsqrt(jnp.sum(f * f, axis=1))
        fn = f * inv_rn[:, None]
        p1 = jax.lax.dot_general(
            fn, m_ref[...], (((1,), (1,)), ((), ())),
            preferred_element_type=jnp.float32)
        acc_ref[...] += _onehot_moments(lab_ref[0, 0, :], p1)

    @pl.when(i == _K)
    def _final():
        accs = acc_ref[...]
        cts = ct_ref[...]
        sw = _N - 2.0 * jnp.sum(accs[0, :]) + jnp.sum(accs[1, :])
        ct = _N - 2.0 * jnp.sum(cts[0, :]) + jnp.sum(cts[1, :])
        loss = (sw / _N) / sc_ref[0] + ct / _N
        out_ref[...] = jnp.full((1, 1), loss, jnp.float32)


def kernel(features, labels, centers):
    part = _sc_segsum(features, labels)
    part3 = part.reshape(_NW, _C, _CW)
    lab3 = labels.reshape(_K, 1, _B)
    ct_acc = pl.pallas_call(
        _ct_body,
        grid=(_K,),
        in_specs=[
            pl.BlockSpec((1, 1, _B), lambda i: (i, 0, 0)),
            pl.BlockSpec((_B, _D), lambda i: (i, 0)),
            pl.BlockSpec((_C, _D), lambda i: (0, 0)),
        ],
        out_specs=pl.BlockSpec((2, _CP), lambda i: (0, 0)),
        out_shape=jax.ShapeDtypeStruct((2, _CP), jnp.float32),
        scratch_shapes=[
            pltpu.VMEM((_CP, _D), jnp.float32),
            pltpu.VMEM((2, _CP), jnp.float32),
        ],
    )(lab3, features, centers)
    out = pl.pallas_call(
        _fin_body,
        grid=(_K + 1,),
        in_specs=[
            pl.BlockSpec((_NW, _C, _CW), lambda i: (0, 0, 0)),
            pl.BlockSpec((1, 1, _B), lambda i: (jnp.maximum(i - 1, 0), 0, 0)),
            pl.BlockSpec((_B, _D), lambda i: (jnp.maximum(i - 1, 0), 0)),
            pl.BlockSpec((2, _CP), lambda i: (0, 0)),
        ],
        out_specs=pl.BlockSpec((1, 1), lambda i: (0, 0)),
        out_shape=jax.ShapeDtypeStruct((1, 1), jnp.float32),
        scratch_shapes=[
            pltpu.VMEM((_CP, _D), jnp.float32),
            pltpu.VMEM((2, _CP), jnp.float32),
            pltpu.SMEM((4,), jnp.float32),
        ],
    )(part3, lab3, features, ct_acc)
    return out.reshape(())


# speedup vs baseline: 1.6953x; 1.2914x over previous
import functools

import jax
import jax.numpy as jnp
from jax import lax
from jax.experimental import pallas as pl
from jax.experimental.pallas import tpu as pltpu
from jax.experimental.pallas import tpu_sc as plsc

_N = 16384
_D = 128
_C = 100
_CP = 128
_CS = 112
_B = 2048
_K = _N // _B
_NW = 32
_RW = _N // _NW
_G = _RW // 128


def _sc_body(feat_hbm, lab_hbm, out_hbm, f_v, lab_v, z_v, acc_sh):
    c = lax.axis_index("c")
    s = lax.axis_index("s")
    wid = c * 16 + s
    pltpu.sync_copy(feat_hbm.at[pl.ds(wid * _RW, _RW), :], f_v)
    pltpu.sync_copy(lab_hbm.at[pl.ds(wid * _G, _G), :], lab_v)

    zeros16 = jnp.zeros((16,), jnp.float32)
    for r in range(_CS // 16):
        for k in range(_D // 16):
            z_v[r, pl.ds(k * 16, 16)] = zeros16
    pltpu.sync_copy(z_v.at[pl.ds(0, _CS // 16), :],
                    acc_sh.at[pl.ds(s * (_CS // 16), _CS // 16), :])
    plsc.subcore_barrier()

    for j in range(_G):
        pltpu.sync_copy(f_v.at[pl.ds(j * 128, 128), :],
                        acc_sh.at[lab_v.at[j]], add=True)
    plsc.subcore_barrier()

    @pl.when(s == 0)
    def _dump():
        pltpu.sync_copy(acc_sh, out_hbm.at[c])


_sc_segsum = functools.partial(
    pl.kernel,
    out_type=jax.ShapeDtypeStruct((2, _CS, _D), jnp.float32),
    mesh=plsc.VectorSubcoreMesh(core_axis_name="c", subcore_axis_name="s"),
    compiler_params=pltpu.CompilerParams(needs_layout_passes=False),
    scratch_types=[
        pltpu.VMEM((_RW, _D), jnp.float32),
        pltpu.VMEM((_G, 128), jnp.int32),
        pltpu.VMEM((_CS // 16, _D), jnp.float32),
        pltpu.VMEM_SHARED((_CS, _D), jnp.float32),
    ],
)(_sc_body)


def _onehot_moments(lab, p):
    oh = (lab[:, None] == jax.lax.broadcasted_iota(jnp.int32, (_B, _CP), 1)
          ).astype(jnp.float32)
    a = oh * p
    return jnp.concatenate([
        jnp.sum(oh, axis=0, keepdims=True),
        jnp.sum(a, axis=0, keepdims=True),
        jnp.sum(a * p, axis=0, keepdims=True),
    ], axis=0)


def _ct_body(lab_ref, feat_ref, cent_ref, out_ref, cb_ref, acc_ref):
    i = pl.program_id(0)

    @pl.when(i == 0)
    def _prep():
        cent = cent_ref[...]
        cbn = jnp.sqrt(jnp.sum(cent * cent, axis=1))
        cb_ref[...] = jnp.concatenate(
            [cent * cbn[:, None], jnp.zeros((_CP - _C, _D), jnp.float32)],
            axis=0)
        acc_ref[...] = jnp.zeros((3, _CP), jnp.float32)

    f = feat_ref[...]
    inv_rn = jax.lax.rsqrt(jnp.sum(f * f, axis=1))
    fn = f * inv_rn[:, None]
    p2 = jax.lax.dot_general(
        fn, cb_ref[...], (((1,), (1,)), ((), ())),
        preferred_element_type=jnp.float32)
    acc_ref[...] += _onehot_moments(lab_ref[0, 0, :], p2)

    @pl.when(i == _K - 1)
    def _emit():
        out_ref[...] = acc_ref[...]


def _fin_body(part_ref, lab_ref, feat_ref, ct_ref, out_ref,
              m_ref, acc_ref, sc_ref):
    i = pl.program_id(0)

    @pl.when(i == 0)
    def _boundary():
        t = part_ref[0, :_C, :] + part_ref[1, :_C, :]
        ni = ct_ref[0, :_C]
        om = jnp.sum(t, axis=0) / _N
        ni_safe = jnp.where(ni > 0, ni, 1.0)
        mc = t / ni_safe[:, None]
        om_norm = jnp.sqrt(jnp.sum(om * om))
        mcn = jnp.sqrt(jnp.sum(mc * mc, axis=1))
        mcn_safe = jnp.where(mcn > 0, mcn, 1.0)
        d_cls = 1.0 - jnp.sum(mc * om[None, :], axis=1) / mcn_safe * om_norm
        sc_ref[0] = jnp.sum((ni / _N) * d_cls * d_cls)
        m = mc * mcn[:, None]
        m_ref[...] = jnp.concatenate(
            [m, jnp.zeros((_CP - _C, _D), jnp.float32)], axis=0)
        acc_ref[...] = jnp.zeros((3, _CP), jnp.float32)

    @pl.when(i > 0)
    def _phase_b():
        f = feat_ref[...]
        inv_rn = jax.lax.rsqrt(jnp.sum(f * f, axis=1))
        fn = f * inv_rn[:, None]
        p1 = jax.lax.dot_general(
            fn, m_ref[...], (((1,), (1,)), ((), ())),
            preferred_element_type=jnp.float32)
        acc_ref[...] += _onehot_moments(lab_ref[0, 0, :], p1)

    @pl.when(i == _K)
    def _final():
        accs = acc_ref[...]
        cts = ct_ref[...]
        sw = _N - 2.0 * jnp.sum(accs[1, :]) + jnp.sum(accs[2, :])
        ct = _N - 2.0 * jnp.sum(cts[1, :]) + jnp.sum(cts[2, :])
        loss = (sw / _N) / sc_ref[0] + ct / _N
        out_ref[...] = jnp.full((1, 1), loss, jnp.float32)


def kernel(features, labels, centers):
    lab2 = labels.reshape(_NW * _G, 128)
    part = _sc_segsum(features, lab2)
    lab3 = labels.reshape(_K, 1, _B)
    ct_acc = pl.pallas_call(
        _ct_body,
        grid=(_K,),
        in_specs=[
            pl.BlockSpec((1, 1, _B), lambda i: (i, 0, 0)),
            pl.BlockSpec((_B, _D), lambda i: (i, 0)),
            pl.BlockSpec((_C, _D), lambda i: (0, 0)),
        ],
        out_specs=pl.BlockSpec((3, _CP), lambda i: (0, 0)),
        out_shape=jax.ShapeDtypeStruct((3, _CP), jnp.float32),
        scratch_shapes=[
            pltpu.VMEM((_CP, _D), jnp.float32),
            pltpu.VMEM((3, _CP), jnp.float32),
        ],
    )(lab3, features, centers)
    out = pl.pallas_call(
        _fin_body,
        grid=(_K + 1,),
        in_specs=[
            pl.BlockSpec((2, _CS, _D), lambda i: (0, 0, 0)),
            pl.BlockSpec((1, 1, _B), lambda i: (jnp.maximum(i - 1, 0), 0, 0)),
            pl.BlockSpec((_B, _D), lambda i: (jnp.maximum(i - 1, 0), 0)),
            pl.BlockSpec((3, _CP), lambda i: (0, 0)),
        ],
        out_specs=pl.BlockSpec((1, 1), lambda i: (0, 0)),
        out_shape=jax.ShapeDtypeStruct((1, 1), jnp.float32),
        scratch_shapes=[
            pltpu.VMEM((_CP, _D), jnp.float32),
            pltpu.VMEM((3, _CP), jnp.float32),
            pltpu.SMEM((4,), jnp.float32),
        ],
    )(part, lab3, features, ct_acc)
    return out.reshape(())


# speedup vs baseline: 1.7340x; 1.0228x over previous
import functools

import jax
import jax.numpy as jnp
from jax import lax
from jax.experimental import pallas as pl
from jax.experimental.pallas import tpu as pltpu
from jax.experimental.pallas import tpu_sc as plsc

_N = 16384
_D = 128
_C = 100
_CP = 128
_CS = 112
_B = 2048
_K = _N // _B
_NW = 32
_RW = _N // _NW
_G = _RW // 128


def _sc_body(feat_hbm, lab_hbm, out_hbm, f_v, lab_v, z_v, acc_sh, sem):
    c = lax.axis_index("c")
    s = lax.axis_index("s")
    wid = c * 16 + s
    fcp = pltpu.make_async_copy(feat_hbm.at[pl.ds(wid * _RW, _RW), :], f_v,
                                sem)
    fcp.start()
    lcp = pltpu.make_async_copy(lab_hbm.at[pl.ds(wid * _G, _G), :], lab_v,
                                sem)
    lcp.start()

    zeros16 = jnp.zeros((16,), jnp.float32)
    for r in range(_CS // 16):
        for k in range(_D // 16):
            z_v[r, pl.ds(k * 16, 16)] = zeros16
    pltpu.sync_copy(z_v.at[pl.ds(0, _CS // 16), :],
                    acc_sh.at[pl.ds(s * (_CS // 16), _CS // 16), :])
    plsc.subcore_barrier()
    fcp.wait()
    lcp.wait()

    descs = []
    for j in range(_G):
        descs.append(pltpu.async_copy(
            f_v.at[pl.ds(j * 128, 128), :], acc_sh.at[lab_v.at[j]], sem,
            add=True))
    for d in descs:
        d.wait()
    plsc.subcore_barrier()

    @pl.when(s == 0)
    def _dump():
        pltpu.sync_copy(acc_sh, out_hbm.at[c])


_sc_segsum = functools.partial(
    pl.kernel,
    out_type=jax.ShapeDtypeStruct((2, _CS, _D), jnp.float32),
    mesh=plsc.VectorSubcoreMesh(core_axis_name="c", subcore_axis_name="s"),
    compiler_params=pltpu.CompilerParams(needs_layout_passes=False),
    scratch_types=[
        pltpu.VMEM((_RW, _D), jnp.float32),
        pltpu.VMEM((_G, 128), jnp.int32),
        pltpu.VMEM((_CS // 16, _D), jnp.float32),
        pltpu.VMEM_SHARED((_CS, _D), jnp.float32),
        pltpu.SemaphoreType.DMA,
    ],
)(_sc_body)


def _onehot_moments(lab, p):
    oh = (lab[:, None] == jax.lax.broadcasted_iota(jnp.int32, (_B, _CP), 1)
          ).astype(jnp.float32)
    a = oh * p
    return jnp.concatenate([
        jnp.sum(oh, axis=0, keepdims=True),
        jnp.sum(a, axis=0, keepdims=True),
        jnp.sum(a * p, axis=0, keepdims=True),
    ], axis=0)


def _ct_body(lab_ref, feat_ref, cent_ref, out_ref, cb_ref, acc_ref):
    i = pl.program_id(0)

    @pl.when(i == 0)
    def _prep():
        cent = cent_ref[...]
        cbn = jnp.sqrt(jnp.sum(cent * cent, axis=1))
        cb_ref[...] = jnp.concatenate(
            [cent * cbn[:, None], jnp.zeros((_CP - _C, _D), jnp.float32)],
            axis=0)
        acc_ref[...] = jnp.zeros((3, _CP), jnp.float32)

    f = feat_ref[...]
    inv_rn = jax.lax.rsqrt(jnp.sum(f * f, axis=1))
    fn = f * inv_rn[:, None]
    p2 = jax.lax.dot_general(
        fn, cb_ref[...], (((1,), (1,)), ((), ())),
        preferred_element_type=jnp.float32)
    acc_ref[...] += _onehot_moments(lab_ref[0, 0, :], p2)

    @pl.when(i == _K - 1)
    def _emit():
        out_ref[...] = acc_ref[...]


def _fin_body(part_ref, lab_ref, feat_ref, ct_ref, out_ref,
              m_ref, acc_ref, sc_ref):
    i = pl.program_id(0)

    @pl.when(i == 0)
    def _boundary():
        t = part_ref[0, :_C, :] + part_ref[1, :_C, :]
        ni = ct_ref[0, :_C]
        om = jnp.sum(t, axis=0) / _N
        ni_safe = jnp.where(ni > 0, ni, 1.0)
        mc = t / ni_safe[:, None]
        om_norm = jnp.sqrt(jnp.sum(om * om))
        mcn = jnp.sqrt(jnp.sum(mc * mc, axis=1))
        mcn_safe = jnp.where(mcn > 0, mcn, 1.0)
        d_cls = 1.0 - jnp.sum(mc * om[None, :], axis=1) / mcn_safe * om_norm
        sc_ref[0] = jnp.sum((ni / _N) * d_cls * d_cls)
        m = mc * mcn[:, None]
        m_ref[...] = jnp.concatenate(
            [m, jnp.zeros((_CP - _C, _D), jnp.float32)], axis=0)
        acc_ref[...] = jnp.zeros((3, _CP), jnp.float32)

    @pl.when(i > 0)
    def _phase_b():
        f = feat_ref[...]
        inv_rn = jax.lax.rsqrt(jnp.sum(f * f, axis=1))
        fn = f * inv_rn[:, None]
        p1 = jax.lax.dot_general(
            fn, m_ref[...], (((1,), (1,)), ((), ())),
            preferred_element_type=jnp.float32)
        acc_ref[...] += _onehot_moments(lab_ref[0, 0, :], p1)

    @pl.when(i == _K)
    def _final():
        accs = acc_ref[...]
        cts = ct_ref[...]
        sw = _N - 2.0 * jnp.sum(accs[1, :]) + jnp.sum(accs[2, :])
        ct = _N - 2.0 * jnp.sum(cts[1, :]) + jnp.sum(cts[2, :])
        loss = (sw / _N) / sc_ref[0] + ct / _N
        out_ref[...] = jnp.full((1, 1), loss, jnp.float32)


def kernel(features, labels, centers):
    lab2 = labels.reshape(_NW * _G, 128)
    part = _sc_segsum(features, lab2)
    lab3 = labels.reshape(_K, 1, _B)
    ct_acc = pl.pallas_call(
        _ct_body,
        grid=(_K,),
        in_specs=[
            pl.BlockSpec((1, 1, _B), lambda i: (i, 0, 0)),
            pl.BlockSpec((_B, _D), lambda i: (i, 0)),
            pl.BlockSpec((_C, _D), lambda i: (0, 0)),
        ],
        out_specs=pl.BlockSpec((3, _CP), lambda i: (0, 0)),
        out_shape=jax.ShapeDtypeStruct((3, _CP), jnp.float32),
        scratch_shapes=[
            pltpu.VMEM((_CP, _D), jnp.float32),
            pltpu.VMEM((3, _CP), jnp.float32),
        ],
    )(lab3, features, centers)
    out = pl.pallas_call(
        _fin_body,
        grid=(_K + 1,),
        in_specs=[
            pl.BlockSpec((2, _CS, _D), lambda i: (0, 0, 0)),
            pl.BlockSpec((1, 1, _B), lambda i: (jnp.maximum(i - 1, 0), 0, 0)),
            pl.BlockSpec((_B, _D), lambda i: (jnp.maximum(i - 1, 0), 0)),
            pl.BlockSpec((3, _CP), lambda i: (0, 0)),
        ],
        out_specs=pl.BlockSpec((1, 1), lambda i: (0, 0)),
        out_shape=jax.ShapeDtypeStruct((1, 1), jnp.float32),
        scratch_shapes=[
            pltpu.VMEM((_CP, _D), jnp.float32),
            pltpu.VMEM((3, _CP), jnp.float32),
            pltpu.SMEM((4,), jnp.float32),
        ],
    )(part, lab3, features, ct_acc)
    return out.reshape(())
